# baseline (speedup 1.0000x reference)
import functools

import jax
import jax.numpy as jnp
from jax import lax
from jax.experimental import pallas as pl
from jax.experimental.pallas import tpu as pltpu
from jax.experimental.pallas import tpu_sc as plsc

NC = 2
NS = 16
NW = NC * NS
LANES = 16
CHUNK = 128
RING = 8
AHEAD = 4


def _make_emb_kernel(n_ids, vocab, d, seq_len):
    assert d % LANES == 0
    per_w = n_ids // NW
    assert per_w * NW == n_ids
    nstep = per_w // CHUNK
    assert nstep * CHUNK == per_w
    ngroup = nstep // RING
    assert ngroup * RING == nstep
    assert per_w % seq_len == 0
    assert seq_len >= CHUNK
    d_vecs = d // LANES

    mesh = plsc.VectorSubcoreMesh(core_axis_name="c", subcore_axis_name="s")
    scratch = (
        [pltpu.VMEM((per_w,), jnp.int32)]
        + [pltpu.VMEM((seq_len + CHUNK, d), jnp.float32)]
        + [pltpu.VMEM((CHUNK, d), jnp.float32) for _ in range(RING)]
        + [pltpu.SemaphoreType.DMA for _ in range(2 * RING)]
    )

    @functools.partial(
        pl.kernel,
        out_type=jax.ShapeDtypeStruct((n_ids, d), jnp.float32),
        mesh=mesh,
        scratch_types=scratch,
        compiler_params=pltpu.CompilerParams(use_tc_tiling_on_sc=False),
    )
    def emb_kernel(ids_hbm, table_hbm, pos_hbm, out_hbm, idx_v, posext_v, *rest):
        rows = rest[:RING]
        gsem = rest[RING:2 * RING]
        ssem = rest[2 * RING:]

        wid = lax.axis_index("s") * NC + lax.axis_index("c")
        base = wid * per_w

        pltpu.sync_copy(ids_hbm.at[pl.ds(base, per_w)], idx_v)
        pltpu.sync_copy(pos_hbm, posext_v.at[pl.ds(0, seq_len)])
        pltpu.sync_copy(pos_hbm.at[pl.ds(0, CHUNK)],
                        posext_v.at[pl.ds(seq_len, CHUNK)])

        def issue_gather(step, buf):
            idx_sl = idx_v.at[pl.ds(step * CHUNK, CHUNK)]
            pltpu.async_copy(table_hbm.at[idx_sl], rows[buf], gsem[buf])

        def wait_gather(step, buf):
            idx_sl = idx_v.at[pl.ds(step * CHUNK, CHUNK)]
            pltpu.make_async_copy(table_hbm.at[idx_sl], rows[buf],
                                  gsem[buf]).wait()

        def issue_store(step, buf):
            off = base + step * CHUNK
            pltpu.async_copy(rows[buf], out_hbm.at[pl.ds(off, CHUNK)],
                             ssem[buf])

        def wait_store(step, buf):
            off = base + step * CHUNK
            pltpu.make_async_copy(rows[buf], out_hbm.at[pl.ds(off, CHUNK)],
                                  ssem[buf]).wait()

        for b in range(AHEAD):
            issue_gather(b, b)

        phase_step = [(CHUNK * j) % seq_len for j in range(RING)]
        phase_inc = (CHUNK * RING) % seq_len

        def group_body(g, s0):
            for j in range(RING):
                s = RING * g + j
                bp = (j + AHEAD) % RING

                if j < AHEAD:
                    @pl.when(g >= 1)
                    def _():
                        wait_store(s - AHEAD, bp)
                    issue_gather(s + AHEAD, bp)
                else:
                    wait_store(s - AHEAD, bp)

                    @pl.when(g <= ngroup - 2)
                    def _():
                        issue_gather(s + AHEAD, bp)

                wait_gather(s, j)

                p = s0 + phase_step[j]
                p = jnp.where(p >= seq_len, p - seq_len, p)

                def add_body(r, carry, j=j, p=p):
                    posr = p + r
                    for c in range(d_vecs):
                        pv = posext_v[posr, pl.ds(LANES * c, LANES)]
                        plsc.addupdate(
                            rows[j].at[r, pl.ds(LANES * c, LANES)], pv)
                    return carry

                lax.fori_loop(0, CHUNK, add_body, 0, unroll=4)

                issue_store(s, j)

            s0 = s0 + phase_inc
            return jnp.where(s0 >= seq_len, s0 - seq_len, s0)

        lax.fori_loop(0, ngroup, group_body, jnp.int32(0))

        for b in range(AHEAD, RING):
            wait_store(nstep - RING + b, b)

    return emb_kernel


def kernel(input_ids, table, pos_table):
    batch, seq_len = input_ids.shape
    vocab, d = table.shape
    n_ids = batch * seq_len
    emb = _make_emb_kernel(n_ids, vocab, d, seq_len)
    out = emb(input_ids.reshape(n_ids), table, pos_table)
    return out.reshape(batch, seq_len, d)
